# baseline (speedup 1.0000x reference)
import functools

import jax
import jax.numpy as jnp
from jax import lax
from jax.experimental import pallas as pl
from jax.experimental.pallas import tpu as pltpu

_VOCAB_PAD = 128
_BB = 64


def _table_body(cbfv_ref, w_ref, b_ref, out_ref):
    out_ref[...] = lax.dot_general(
        cbfv_ref[...], w_ref[...], (((1,), (1,)), ((), ())),
        preferred_element_type=jnp.float32) + b_ref[...]


def _make_expand_body(t, d_model):
    tb = _BB * t

    def _expand_body(idx_ref, table_ref, out_ref):
        ids = idx_ref[0, 0, :]
        onehot = (ids[:, None]
                  == lax.broadcasted_iota(jnp.int32, (tb, _VOCAB_PAD), 1))
        y = lax.dot_general(
            onehot.astype(jnp.float32), table_ref[...],
            (((1,), (0,)), ((), ())), preferred_element_type=jnp.float32)
        for g in range(_BB):
            out_ref[g] = y[g * t:(g + 1) * t, :]

    return _expand_body


@functools.cache
def _make_expand(bsz, t, d_model):
    n_blk = bsz // _BB
    tb = _BB * t
    return pl.pallas_call(
        _make_expand_body(t, d_model),
        grid=(n_blk,),
        in_specs=[
            pl.BlockSpec((1, 1, tb), lambda i: (i, 0, 0)),
            pl.BlockSpec((_VOCAB_PAD, d_model), lambda i: (0, 0)),
        ],
        out_specs=pl.BlockSpec((_BB, t, d_model), lambda i: (i, 0, 0)),
        out_shape=jax.ShapeDtypeStruct((bsz, t, d_model), jnp.float32),
        compiler_params=pltpu.CompilerParams(
            dimension_semantics=("arbitrary",)),
    )


def kernel(src, cbfv, W, b):
    bsz, t = src.shape
    d_model = W.shape[0]
    cbfv_pad = jnp.pad(cbfv, ((0, _VOCAB_PAD - cbfv.shape[0]), (0, 0)))
    table = pl.pallas_call(
        _table_body,
        out_shape=jax.ShapeDtypeStruct((_VOCAB_PAD, d_model), jnp.float32),
    )(cbfv_pad, W, b.reshape(1, d_model))

    idx = src.reshape(bsz // _BB, 1, _BB * t).astype(jnp.int32)
    return _make_expand(bsz, t, d_model)(idx, table)
